# baseline (speedup 1.0000x reference)
import functools

import jax
import jax.numpy as jnp
from jax.experimental import pallas as pl

F32 = jnp.float32


def _maxpool_k3s2(a, valid_h, valid_w):
    oh = (valid_h - 1) // 2
    ow = (valid_w - 1) // 2
    W = a.shape[1]
    C = a.shape[2]
    p = a[: 2 * oh].reshape(oh, 2, W, C).max(axis=1)
    q = a[1 : 2 * oh + 1].reshape(oh, 2, W, C)[:, 1]
    hh = jnp.maximum(p, q)
    p = hh[:, : 2 * ow].reshape(oh, ow, 2, C).max(axis=2)
    q = hh[:, 1 : 2 * ow + 1].reshape(oh, ow, 2, C)[:, :, 1]
    return jnp.maximum(p, q)


def _shift_conv(xpad_flat, w, out_h, wp, ksize):
    m = out_h * wp
    cout = w.shape[-1]
    acc = jnp.zeros((m, cout), dtype=F32)
    for di in range(ksize):
        for dj in range(ksize):
            s = di * wp + dj
            acc = acc + jnp.dot(
                xpad_flat[s : s + m, :], w[di * ksize + dj],
                preferred_element_type=F32)
    return acc


def _front_body(x_ref, w1_ref, b1_ref, w2_ref, b2_ref, out_ref):
    xf = x_ref[0]
    acc = _shift_conv(xf, w1_ref[...], out_h=55, wp=57, ksize=3)
    a = acc.reshape(55, 57, 96) + b1_ref[...][None, :, :].reshape(1, 1, 96)
    h1 = _maxpool_k3s2(a, valid_h=55, valid_w=55)
    z2r = jnp.zeros((2, 27, 96), F32)
    h1p = jnp.concatenate([z2r, h1, z2r, jnp.zeros((1, 27, 96), F32)], axis=0)
    z2c = jnp.zeros((32, 2, 96), F32)
    h1p = jnp.concatenate([z2c, h1p, z2c], axis=1)
    xf2 = h1p.reshape(32 * 31, 96)
    acc2 = _shift_conv(xf2, w2_ref[...], out_h=27, wp=31, ksize=5)
    b = acc2.reshape(27, 31, 256) + b2_ref[...].reshape(1, 1, 256)
    out_ref[0] = _maxpool_k3s2(b, valid_h=27, valid_w=27)


def _pad_1_and_flatten(v, c):
    zr = jnp.zeros((1, 13, c), F32)
    vp = jnp.concatenate([zr, v, zr, zr], axis=0)
    zc = jnp.zeros((16, 1, c), F32)
    vp = jnp.concatenate([zc, vp, zc], axis=1)
    return vp.reshape(240, c)


def _back_body(x_ref, w3_ref, b3_ref, w4_ref, b4_ref, w5_ref, b5_ref,
               out_ref):
    x = x_ref[0]
    acc = _shift_conv(_pad_1_and_flatten(x, 256), w3_ref[...], 13, 15, 3)
    c3 = acc.reshape(13, 15, 384) + b3_ref[...].reshape(1, 1, 384)
    acc = _shift_conv(_pad_1_and_flatten(c3[:, :13], 384), w4_ref[...], 13,
                      15, 3)
    c4 = acc.reshape(13, 15, 384) + b4_ref[...].reshape(1, 1, 384)
    acc = _shift_conv(_pad_1_and_flatten(c4[:, :13], 384), w5_ref[...], 13,
                      15, 3)
    c5 = acc.reshape(13, 15, 256) + b5_ref[...].reshape(1, 1, 256)
    out_ref[0] = _maxpool_k3s2(c5, valid_h=13, valid_w=13)


def _fc_body(w_ref, x_ref, b_ref, out_ref):
    out_ref[...] = (
        jnp.dot(w_ref[...], x_ref[...], preferred_element_type=F32)
        + b_ref[...])


def _fc(w, xt, b, n_block):
    n, k = w.shape
    cols = xt.shape[1]
    grid = n // n_block
    return pl.pallas_call(
        _fc_body,
        grid=(grid,),
        in_specs=[
            pl.BlockSpec((n_block, k), lambda i: (i, 0)),
            pl.BlockSpec((k, cols), lambda i: (0, 0)),
            pl.BlockSpec((n_block, 1), lambda i: (i, 0)),
        ],
        out_specs=pl.BlockSpec((n_block, cols), lambda i: (i, 0)),
        out_shape=jax.ShapeDtypeStruct((n, cols), F32),
    )(w, xt, b.reshape(n, 1))


@jax.jit
def kernel(x, W1, b1, W2, b2, W3, b3, W4, b4, W5, b5, W6, b6, W7, b7, W8, b8):
    batch = x.shape[0]

    xp = jnp.pad(x, ((0, 0), (0, 0), (0, 1), (0, 1)))
    xs = xp.reshape(batch, 3, 57, 4, 57, 4).transpose(0, 2, 4, 1, 3, 5)
    xs = xs.reshape(batch, 57, 57, 48)
    xs = jnp.pad(xs, ((0, 0), (0, 1), (0, 0), (0, 0)))
    xs = xs.reshape(batch, 58 * 57, 48)

    w1p = jnp.pad(W1, ((0, 0), (0, 0), (0, 1), (0, 1)))
    w1s = w1p.reshape(96, 3, 3, 4, 3, 4).transpose(2, 4, 1, 3, 5, 0)
    w1s = w1s.reshape(9, 48, 96)

    def conv_w(w):
        k = w.shape[-1]
        return w.transpose(2, 3, 1, 0).reshape(k * k, w.shape[1], w.shape[0])

    w2s, w3s, w4s, w5s = conv_w(W2), conv_w(W3), conv_w(W4), conv_w(W5)

    h = pl.pallas_call(
        _front_body,
        grid=(batch,),
        in_specs=[
            pl.BlockSpec((1, 58 * 57, 48), lambda n: (n, 0, 0)),
            pl.BlockSpec((9, 48, 96), lambda n: (0, 0, 0)),
            pl.BlockSpec((1, 96), lambda n: (0, 0)),
            pl.BlockSpec((25, 96, 256), lambda n: (0, 0, 0)),
            pl.BlockSpec((1, 256), lambda n: (0, 0)),
        ],
        out_specs=pl.BlockSpec((1, 13, 13, 256), lambda n: (n, 0, 0, 0)),
        out_shape=jax.ShapeDtypeStruct((batch, 13, 13, 256), F32),
    )(xs, w1s, b1.reshape(1, 96), w2s, b2.reshape(1, 256))

    h = pl.pallas_call(
        _back_body,
        grid=(batch,),
        in_specs=[
            pl.BlockSpec((1, 13, 13, 256), lambda n: (n, 0, 0, 0)),
            pl.BlockSpec((9, 256, 384), lambda n: (0, 0, 0)),
            pl.BlockSpec((1, 384), lambda n: (0, 0)),
            pl.BlockSpec((9, 384, 384), lambda n: (0, 0, 0)),
            pl.BlockSpec((1, 384), lambda n: (0, 0)),
            pl.BlockSpec((9, 384, 256), lambda n: (0, 0, 0)),
            pl.BlockSpec((1, 256), lambda n: (0, 0)),
        ],
        out_specs=pl.BlockSpec((1, 6, 6, 256), lambda n: (n, 0, 0, 0)),
        out_shape=jax.ShapeDtypeStruct((batch, 6, 6, 256), F32),
    )(h, w3s, b3.reshape(1, 384), w4s, b4.reshape(1, 384), w5s,
      b5.reshape(1, 256))

    xt = h.transpose(3, 1, 2, 0).reshape(9216, batch)
    ht = _fc(W6, xt, b6, n_block=512)
    ht = _fc(W7, ht, b7, n_block=512)
    ht = _fc(W8, ht, b8, n_block=200)
    return ht.T


# speedup vs baseline: 1.1865x; 1.1865x over previous
import functools

import jax
import jax.numpy as jnp
from jax.experimental import pallas as pl

F32 = jnp.float32


def _maxpool_k3s2(a, valid_h, valid_w):
    oh = (valid_h - 1) // 2
    ow = (valid_w - 1) // 2
    W = a.shape[1]
    C = a.shape[2]
    p = a[: 2 * oh].reshape(oh, 2, W, C).max(axis=1)
    q = a[1 : 2 * oh + 1].reshape(oh, 2, W, C)[:, 1]
    hh = jnp.maximum(p, q)
    p = hh[:, : 2 * ow].reshape(oh, ow, 2, C).max(axis=2)
    q = hh[:, 1 : 2 * ow + 1].reshape(oh, ow, 2, C)[:, :, 1]
    return jnp.maximum(p, q)


def _packed_conv(xpad_flat, wcat, out_h, wp, ksize):
    m = out_h * wp
    cout = wcat.shape[-1]
    span = (ksize - 1) * wp + m
    xcol = jnp.concatenate(
        [xpad_flat[dj : dj + span, :] for dj in range(ksize)], axis=1)
    acc = jnp.zeros((m, cout), dtype=F32)
    for di in range(ksize):
        acc = acc + jnp.dot(
            xcol[di * wp : di * wp + m, :], wcat[di],
            preferred_element_type=F32)
    return acc


def _front_body(x_ref, w1_ref, b1_ref, w2_ref, b2_ref, out_ref):
    xf = x_ref[0].astype(jnp.bfloat16)
    xcol = jnp.concatenate(
        [xf[di * 57 + dj : di * 57 + dj + 3135, :]
         for di in range(3) for dj in range(3)], axis=1)
    acc = jnp.dot(xcol, w1_ref[...], preferred_element_type=F32)
    a = acc.reshape(55, 57, 96) + b1_ref[...].reshape(1, 1, 96)
    h1 = _maxpool_k3s2(a, valid_h=55, valid_w=55)
    h1 = h1.astype(jnp.bfloat16)
    z2r = jnp.zeros((2, 27, 96), jnp.bfloat16)
    h1p = jnp.concatenate(
        [z2r, h1, z2r, jnp.zeros((1, 27, 96), jnp.bfloat16)], axis=0)
    z2c = jnp.zeros((32, 2, 96), jnp.bfloat16)
    h1p = jnp.concatenate([z2c, h1p, z2c], axis=1)
    xf2 = h1p.reshape(32 * 31, 96)
    acc2 = _packed_conv(xf2, w2_ref[...], out_h=27, wp=31, ksize=5)
    b = acc2.reshape(27, 31, 256) + b2_ref[...].reshape(1, 1, 256)
    out_ref[0] = _maxpool_k3s2(b, valid_h=27, valid_w=27)


def _pad_1_and_flatten(v, c):
    bf = jnp.bfloat16
    zr = jnp.zeros((1, 13, c), bf)
    vp = jnp.concatenate([zr, v, zr, zr], axis=0)
    zc = jnp.zeros((16, 1, c), bf)
    vp = jnp.concatenate([zc, vp, zc], axis=1)
    return vp.reshape(240, c)


def _back_body(x_ref, w3_ref, b3_ref, w4_ref, b4_ref, w5_ref, b5_ref,
               out_ref):
    x = x_ref[0].astype(jnp.bfloat16)
    acc = _packed_conv(_pad_1_and_flatten(x, 256), w3_ref[...], 13, 15, 3)
    c3 = acc.reshape(13, 15, 384) + b3_ref[...].reshape(1, 1, 384)
    c3 = c3.astype(jnp.bfloat16)
    acc = _packed_conv(_pad_1_and_flatten(c3[:, :13], 384), w4_ref[...], 13,
                       15, 3)
    c4 = acc.reshape(13, 15, 384) + b4_ref[...].reshape(1, 1, 384)
    c4 = c4.astype(jnp.bfloat16)
    acc = _packed_conv(_pad_1_and_flatten(c4[:, :13], 384), w5_ref[...], 13,
                       15, 3)
    c5 = acc.reshape(13, 15, 256) + b5_ref[...].reshape(1, 1, 256)
    out_ref[0] = _maxpool_k3s2(c5, valid_h=13, valid_w=13)


def _fc_body(w_ref, x_ref, b_ref, out_ref):
    out_ref[...] = (
        jnp.dot(w_ref[...].astype(jnp.bfloat16),
                x_ref[...].astype(jnp.bfloat16),
                preferred_element_type=F32)
        + b_ref[...])


def _fc(w, xt, b, n_block):
    n, k = w.shape
    cols = xt.shape[1]
    grid = n // n_block
    return pl.pallas_call(
        _fc_body,
        grid=(grid,),
        in_specs=[
            pl.BlockSpec((n_block, k), lambda i: (i, 0)),
            pl.BlockSpec((k, cols), lambda i: (0, 0)),
            pl.BlockSpec((n_block, 1), lambda i: (i, 0)),
        ],
        out_specs=pl.BlockSpec((n_block, cols), lambda i: (i, 0)),
        out_shape=jax.ShapeDtypeStruct((n, cols), F32),
    )(w, xt, b.reshape(n, 1))


@jax.jit
def kernel(x, W1, b1, W2, b2, W3, b3, W4, b4, W5, b5, W6, b6, W7, b7, W8, b8):
    batch = x.shape[0]

    xp = jnp.pad(x, ((0, 0), (0, 0), (0, 1), (0, 1)))
    xs = xp.reshape(batch, 3, 57, 4, 57, 4).transpose(0, 2, 4, 1, 3, 5)
    xs = xs.reshape(batch, 57, 57, 48)
    xs = jnp.pad(xs, ((0, 0), (0, 1), (0, 0), (0, 0)))
    xs = xs.reshape(batch, 58 * 57, 48)

    w1p = jnp.pad(W1, ((0, 0), (0, 0), (0, 1), (0, 1)))
    w1s = w1p.reshape(96, 3, 3, 4, 3, 4).transpose(2, 4, 1, 3, 5, 0)
    w1s = w1s.reshape(9 * 48, 96).astype(jnp.bfloat16)

    def conv_w(w):
        k = w.shape[-1]
        return (w.transpose(2, 3, 1, 0)
                .reshape(k, k * w.shape[1], w.shape[0]).astype(jnp.bfloat16))

    w2s, w3s, w4s, w5s = conv_w(W2), conv_w(W3), conv_w(W4), conv_w(W5)

    h = pl.pallas_call(
        _front_body,
        grid=(batch,),
        in_specs=[
            pl.BlockSpec((1, 58 * 57, 48), lambda n: (n, 0, 0)),
            pl.BlockSpec((9 * 48, 96), lambda n: (0, 0)),
            pl.BlockSpec((1, 96), lambda n: (0, 0)),
            pl.BlockSpec((5, 5 * 96, 256), lambda n: (0, 0, 0)),
            pl.BlockSpec((1, 256), lambda n: (0, 0)),
        ],
        out_specs=pl.BlockSpec((1, 13, 13, 256), lambda n: (n, 0, 0, 0)),
        out_shape=jax.ShapeDtypeStruct((batch, 13, 13, 256), F32),
    )(xs, w1s, b1.reshape(1, 96), w2s, b2.reshape(1, 256))

    h = pl.pallas_call(
        _back_body,
        grid=(batch,),
        in_specs=[
            pl.BlockSpec((1, 13, 13, 256), lambda n: (n, 0, 0, 0)),
            pl.BlockSpec((3, 3 * 256, 384), lambda n: (0, 0, 0)),
            pl.BlockSpec((1, 384), lambda n: (0, 0)),
            pl.BlockSpec((3, 3 * 384, 384), lambda n: (0, 0, 0)),
            pl.BlockSpec((1, 384), lambda n: (0, 0)),
            pl.BlockSpec((3, 3 * 384, 256), lambda n: (0, 0, 0)),
            pl.BlockSpec((1, 256), lambda n: (0, 0)),
        ],
        out_specs=pl.BlockSpec((1, 6, 6, 256), lambda n: (n, 0, 0, 0)),
        out_shape=jax.ShapeDtypeStruct((batch, 6, 6, 256), F32),
    )(h, w3s, b3.reshape(1, 384), w4s, b4.reshape(1, 384), w5s,
      b5.reshape(1, 256))

    xt = h.transpose(3, 1, 2, 0).reshape(9216, batch)
    ht = _fc(W6, xt, b6, n_block=512)
    ht = _fc(W7, ht, b7, n_block=512)
    ht = _fc(W8, ht, b8, n_block=200)
    return ht.T


# speedup vs baseline: 1.5940x; 1.3435x over previous
import jax
import jax.numpy as jnp
from jax.experimental import pallas as pl

F32 = jnp.float32
BF16 = jnp.bfloat16


def _pool(v, oh, ow2):
    h2, w, c = 2 * oh, v.shape[1], v.shape[2]
    p = v[:h2].reshape(oh, 2, w, c).max(axis=1)
    q = v[1 : h2 + 1].reshape(oh, 2, w, c)[:, 1]
    hh = jnp.maximum(p, q)
    pv = hh.reshape(oh, ow2, 2, c)
    p2 = pv.max(axis=2)
    pp = pv[:, :, 0]
    q2 = jnp.concatenate([pp[:, 1:], pp[:, :1]], axis=1)
    return jnp.maximum(p2, q2)


def _dj_pack(xext, ksize, span, start):
    return jnp.concatenate(
        [xext[start + dj : start + dj + span, :] for dj in range(ksize)],
        axis=1)


def _di_dots(xcol, wcat, ksize, wp, m):
    acc = jnp.dot(xcol[0:m, :], wcat[0], preferred_element_type=F32)
    for di in range(1, ksize):
        acc = acc + jnp.dot(xcol[di * wp : di * wp + m, :], wcat[di],
                            preferred_element_type=F32)
    return acc


def _front_body(x_ref, w1_ref, b1_ref, w2_ref, b2_ref, out_ref):
    xf = x_ref[0]
    xcol = _dj_pack(xf, 3, 3648, 0)
    acc = _di_dots(xcol, w1_ref[...], 3, 64, 3520)
    a = acc.reshape(55, 64, 96) + b1_ref[...].reshape(1, 1, 96)
    h1 = _pool(a, 27, 32)

    h1 = h1.astype(BF16)
    col = jax.lax.broadcasted_iota(jnp.int32, (27, 32, 96), 1)
    h1 = jnp.where(col < 27, h1, jnp.zeros((), BF16)).reshape(864, 96)
    xext = jnp.concatenate(
        [jnp.zeros((72, 96), BF16), h1, jnp.zeros((72, 96), BF16)], axis=0)
    xcol = _dj_pack(xext, 5, 992, 6)
    acc = _di_dots(xcol, w2_ref[...], 5, 32, 864)
    b = acc.reshape(27, 32, 256) + b2_ref[...].reshape(1, 1, 256)
    out_ref[0] = _pool(b, 13, 16)


def _conv3x3(v, wcat, bias, cin):
    r = jax.lax.broadcasted_iota(jnp.int32, (208, cin), 0)
    vm = jnp.where((r % 16) < 13, v, jnp.zeros((), BF16))
    xext = jnp.concatenate(
        [jnp.zeros((24, cin), BF16), vm, jnp.zeros((24, cin), BF16)], axis=0)
    xcol = _dj_pack(xext, 3, 240, 7)
    acc = _di_dots(xcol, wcat, 3, 16, 208)
    return acc + bias


def _back_body(x_ref, w3_ref, b3_ref, w4_ref, b4_ref, w5_ref, b5_ref,
               out_ref):
    x = x_ref[0].astype(BF16).reshape(208, 256)
    c3 = _conv3x3(x, w3_ref[...], b3_ref[...].reshape(1, 384), 256)
    c4 = _conv3x3(c3.astype(BF16), w4_ref[...], b4_ref[...].reshape(1, 384),
                  384)
    c5 = _conv3x3(c4.astype(BF16), w5_ref[...], b5_ref[...].reshape(1, 256),
                  384)
    out_ref[0] = _pool(c5.reshape(13, 16, 256), 6, 8)


def _fc_body(w_ref, x_ref, b_ref, out_ref):
    out_ref[...] = (
        jnp.dot(w_ref[...].astype(BF16), x_ref[...],
                preferred_element_type=F32)
        + b_ref[...])


def _fc(w, xt, b, n_block):
    n, k = w.shape
    cols = xt.shape[1]
    return pl.pallas_call(
        _fc_body,
        grid=(n // n_block,),
        in_specs=[
            pl.BlockSpec((n_block, k), lambda i: (i, 0)),
            pl.BlockSpec((k, cols), lambda i: (0, 0)),
            pl.BlockSpec((n_block, 1), lambda i: (i, 0)),
        ],
        out_specs=pl.BlockSpec((n_block, cols), lambda i: (i, 0)),
        out_shape=jax.ShapeDtypeStruct((n, cols), F32),
    )(w, xt, b.reshape(n, 1))


@jax.jit
def kernel(x, W1, b1, W2, b2, W3, b3, W4, b4, W5, b5, W6, b6, W7, b7, W8, b8):
    batch = x.shape[0]

    xp = jnp.pad(x.astype(BF16), ((0, 0), (0, 0), (0, 5), (0, 29)))
    xs = xp.reshape(batch, 3, 58, 4, 64, 4).transpose(0, 2, 4, 1, 3, 5)
    xs = xs.reshape(batch, 58 * 64, 48)

    w1p = jnp.pad(W1.astype(BF16), ((0, 0), (0, 0), (0, 1), (0, 1)))
    w1s = w1p.reshape(96, 3, 3, 4, 3, 4).transpose(2, 4, 1, 3, 5, 0)
    w1s = w1s.reshape(3, 3 * 48, 96)

    def conv_w(w):
        k = w.shape[-1]
        return (w.astype(BF16).transpose(2, 3, 1, 0)
                .reshape(k, k * w.shape[1], w.shape[0]))

    w2s, w3s, w4s, w5s = conv_w(W2), conv_w(W3), conv_w(W4), conv_w(W5)

    h = pl.pallas_call(
        _front_body,
        grid=(batch,),
        in_specs=[
            pl.BlockSpec((1, 58 * 64, 48), lambda n: (n, 0, 0)),
            pl.BlockSpec((3, 3 * 48, 96), lambda n: (0, 0, 0)),
            pl.BlockSpec((1, 96), lambda n: (0, 0)),
            pl.BlockSpec((5, 5 * 96, 256), lambda n: (0, 0, 0)),
            pl.BlockSpec((1, 256), lambda n: (0, 0)),
        ],
        out_specs=pl.BlockSpec((1, 13, 16, 256), lambda n: (n, 0, 0, 0)),
        out_shape=jax.ShapeDtypeStruct((batch, 13, 16, 256), F32),
    )(xs, w1s, b1.reshape(1, 96), w2s, b2.reshape(1, 256))

    h = pl.pallas_call(
        _back_body,
        grid=(batch,),
        in_specs=[
            pl.BlockSpec((1, 13, 16, 256), lambda n: (n, 0, 0, 0)),
            pl.BlockSpec((3, 3 * 256, 384), lambda n: (0, 0, 0)),
            pl.BlockSpec((1, 384), lambda n: (0, 0)),
            pl.BlockSpec((3, 3 * 384, 384), lambda n: (0, 0, 0)),
            pl.BlockSpec((1, 384), lambda n: (0, 0)),
            pl.BlockSpec((3, 3 * 384, 256), lambda n: (0, 0, 0)),
            pl.BlockSpec((1, 256), lambda n: (0, 0)),
        ],
        out_specs=pl.BlockSpec((1, 6, 8, 256), lambda n: (n, 0, 0, 0)),
        out_shape=jax.ShapeDtypeStruct((batch, 6, 8, 256), F32),
    )(h, w3s, b3.reshape(1, 384), w4s, b4.reshape(1, 384), w5s,
      b5.reshape(1, 256))

    xt = h[:, :, :6, :].transpose(3, 1, 2, 0).reshape(9216, batch)
    xt = xt.astype(BF16)
    ht = _fc(W6, xt, b6, n_block=512)
    ht = _fc(W7, ht.astype(BF16), b7, n_block=512)
    ht = _fc(W8, ht.astype(BF16), b8, n_block=200)
    return ht.T
